# baseline (speedup 1.0000x reference)
import jax
import jax.numpy as jnp
from jax import lax
from jax.experimental import pallas as pl
from jax.experimental.pallas import tpu as pltpu
from jax.experimental.pallas import tpu_sc as plsc

B0, B1 = 64, 32768
M = B0 * B1
JDIM, IDIM = 5, 4
E = JDIM * IDIM
K = 3
NC, NS, L = 2, 16, 16
NW = NC * NS
TG = B1 // 128
BG = 8
NBG = B0 // BG
TGB = 4
NCH = NBG * (TG // TGB)
CPW = NCH // NW
CTOK = BG * TGB * 128
GROUPS = CTOK // L
PLANE = TGB * BG * 128


def _round_bf16(v):
    u = plsc.bitcast(v, jnp.int32)
    u = (u + 0x8000) & jnp.int32(-65536)
    return plsc.bitcast(u, jnp.float32)


def _sc_body(xf, wf, wbf, bf, vals, idxs,
             w_v, wb_v, b_v, in_v, vo_v, io_v, sem):
    cid = lax.axis_index("c")
    sid = lax.axis_index("s")
    wid = sid * NC + cid
    pltpu.sync_copy(wf, w_v)
    pltpu.sync_copy(wbf, wb_v)
    pltpu.sync_copy(bf, b_v)
    lanes = lax.iota(jnp.int32, L)
    wsf = [w_v[pl.ds(k * L, L)] for k in range(E)]
    wsb = [wb_v[pl.ds(k * L, L)] for k in range(E)]
    bs = [b_v[pl.ds(o * L, L)] for o in range(JDIM)]

    def chunk_body(c, carry):
        ci = wid * CPW + c
        bg = ci // (TG // TGB)
        tg0 = (ci % (TG // TGB)) * TGB
        copies = [pltpu.async_copy(
                      xf.at[pl.ds((bg * BG + bl) * JDIM, JDIM),
                            pl.ds(tg0, TGB)],
                      in_v.at[bl], sem)
                  for bl in range(BG)]
        for cp in copies:
            cp.wait()

        def group_body(g, carry):
            bl = g // 32
            r = g - bl * 32
            tgl = r // 8
            g16 = r - tgl * 8
            toff = g16 * L
            obase = tgl * (BG * 128) + bl * 128 + g16 * L
            s = []
            for j in range(JDIM):
                h = [_round_bf16(
                        in_v[bl, j, tgl, pl.ds(i * 128 + toff, L)]
                        + wsf[j * IDIM + i])
                     for i in range(IDIM)]
                acc_sum = None
                for o in range(JDIM):
                    acc = bs[o]
                    for i in range(IDIM):
                        acc = acc + h[i] * wsb[o * IDIM + i]
                    acc = jnp.maximum(acc, 0.0)
                    acc_sum = acc if acc_sum is None else acc_sum + acc
                s.append(acc_sum)
            for k in range(K):
                bv = s[0]
                bi = jnp.zeros((L,), jnp.int32)
                for j in range(1, JDIM):
                    gt = s[j] > bv
                    bv = jnp.where(gt, s[j], bv)
                    bi = jnp.where(gt, j, bi)
                vo_v[k, pl.ds(obase, L)] = bv
                io_v[k, pl.ds(obase, L)] = bi
                if k < K - 1:
                    s = [jnp.where(bi == j, -1.0, s[j]) for j in range(JDIM)]
            return carry

        lax.fori_loop(0, GROUPS, group_body, 0)
        dst = bg * (TG * BG * 128) + tg0 * (BG * 128)
        oc = [pltpu.async_copy(vo_v, vals.at[:, pl.ds(dst, PLANE)], sem),
              pltpu.async_copy(io_v, idxs.at[:, pl.ds(dst, PLANE)], sem)]
        for cp in oc:
            cp.wait()
        return carry

    lax.fori_loop(0, CPW, chunk_body, 0)


def kernel(x, W, b):
    xf = (x.transpose(0, 2, 3, 1)
           .reshape(B0, JDIM, IDIM, TG, 128)
           .transpose(0, 1, 3, 2, 4)
           .reshape(B0 * JDIM, TG, IDIM * 128))
    wi = lax.bitcast_convert_type(W, jnp.int32)
    wi = (wi + 0x7FFF + ((wi >> 16) & 1)) & jnp.int32(-65536)
    Wb = lax.bitcast_convert_type(wi, jnp.float32)
    wf = jnp.repeat(W.reshape(E), L)
    wbf = jnp.repeat(Wb.reshape(E), L)
    bf = jnp.repeat(b, L)
    mesh = plsc.VectorSubcoreMesh(core_axis_name="c", subcore_axis_name="s")
    vals, idxs = pl.kernel(
        _sc_body,
        out_type=(jax.ShapeDtypeStruct((K, M), jnp.float32),
                  jax.ShapeDtypeStruct((K, M), jnp.int32)),
        mesh=mesh,
        compiler_params=pltpu.CompilerParams(needs_layout_passes=False),
        scratch_types=[
            pltpu.VMEM((E * L,), jnp.float32),
            pltpu.VMEM((E * L,), jnp.float32),
            pltpu.VMEM((JDIM * L,), jnp.float32),
            pltpu.VMEM((BG, JDIM, TGB, 512), jnp.float32),
            pltpu.VMEM((K, PLANE), jnp.float32),
            pltpu.VMEM((K, PLANE), jnp.int32),
            pltpu.SemaphoreType.DMA,
        ],
    )(xf, wf, wbf, bf)
    vals = (vals.reshape(K, NBG, TG, BG, 128)
                .transpose(1, 3, 2, 4, 0).reshape(B0, B1, K))
    idxs = (idxs.reshape(K, NBG, TG, BG, 128)
                .transpose(1, 3, 2, 4, 0).reshape(B0, B1, K))
    return vals, idxs


# speedup vs baseline: 1.5197x; 1.5197x over previous
import jax
import jax.numpy as jnp
from jax import lax
from jax.experimental import pallas as pl
from jax.experimental.pallas import tpu as pltpu
from jax.experimental.pallas import tpu_sc as plsc

B0, B1 = 64, 32768
M = B0 * B1
JDIM, IDIM = 5, 4
E = JDIM * IDIM
K = 3
NC, NS, L = 2, 16, 16
NW = NC * NS
TG = B1 // 128
BG = 8
NBG = B0 // BG
TGB = 4
NCH = NBG * (TG // TGB)
CPW = NCH // NW
CTOK = BG * TGB * 128
GROUPS = CTOK // L
PLANE = TGB * BG * 128


def _round_bf16(v):
    u = plsc.bitcast(v, jnp.int32)
    u = (u + 0x8000) & jnp.int32(-65536)
    return plsc.bitcast(u, jnp.float32)


def _sc_body(xf, wf, wbf, bf, vals, idxs,
             w_v, wb_v, b_v, in_v, vo_v, io_v, sem):
    cid = lax.axis_index("c")
    sid = lax.axis_index("s")
    wid = sid * NC + cid
    pltpu.sync_copy(wf, w_v)
    pltpu.sync_copy(wbf, wb_v)
    pltpu.sync_copy(bf, b_v)
    lanes = lax.iota(jnp.int32, L)
    wsf = [w_v[pl.ds(k * L, L)] for k in range(E)]
    wsb = [wb_v[pl.ds(k * L, L)] for k in range(E)]
    bs = [b_v[pl.ds(o * L, L)] for o in range(JDIM)]

    def chunk_body(c, carry):
        ci = wid * CPW + c
        bg = ci // (TG // TGB)
        tg0 = (ci % (TG // TGB)) * TGB
        copies = [pltpu.async_copy(
                      xf.at[pl.ds((((bg * BG + bl) * JDIM + j) * TG + tg0)
                                  * 512, TGB * 512)],
                      in_v.at[pl.ds((bl * JDIM + j) * (TGB * 512),
                                    TGB * 512)], sem)
                  for bl in range(BG) for j in range(JDIM)]
        for cp in copies:
            cp.wait()

        def group_body(g, carry):
            bl = g // 32
            r = g - bl * 32
            tgl = r // 8
            g16 = r - tgl * 8
            toff = g16 * L
            obase = tgl * (BG * 128) + bl * 128 + g16 * L
            s = []
            for j in range(JDIM):
                h = [_round_bf16(
                        in_v[pl.ds((bl * JDIM + j) * (TGB * 512)
                                   + tgl * 512 + i * 128 + toff, L)]
                        + wsf[j * IDIM + i])
                     for i in range(IDIM)]
                acc_sum = None
                for o in range(JDIM):
                    acc = bs[o]
                    for i in range(IDIM):
                        acc = acc + h[i] * wsb[o * IDIM + i]
                    acc = jnp.maximum(acc, 0.0)
                    acc_sum = acc if acc_sum is None else acc_sum + acc
                s.append(acc_sum)
            for k in range(K):
                bv = s[0]
                bi = jnp.zeros((L,), jnp.int32)
                for j in range(1, JDIM):
                    gt = s[j] > bv
                    bv = jnp.where(gt, s[j], bv)
                    bi = jnp.where(gt, j, bi)
                vo_v[pl.ds(k * PLANE + obase, L)] = bv
                io_v[pl.ds(k * PLANE + obase, L)] = bi
                if k < K - 1:
                    s = [jnp.where(bi == j, -1.0, s[j]) for j in range(JDIM)]
            return carry

        lax.fori_loop(0, GROUPS, group_body, 0)
        dst = bg * (TG * BG * 128) + tg0 * (BG * 128)
        oc = [pltpu.async_copy(vo_v.at[pl.ds(k * PLANE, PLANE)],
                               vals.at[pl.ds(k * M + dst, PLANE)], sem)
              for k in range(K)]
        oc += [pltpu.async_copy(io_v.at[pl.ds(k * PLANE, PLANE)],
                                idxs.at[pl.ds(k * M + dst, PLANE)], sem)
               for k in range(K)]
        for cp in oc:
            cp.wait()
        return carry

    lax.fori_loop(0, CPW, chunk_body, 0)


def kernel(x, W, b):
    xf = (x.transpose(0, 2, 3, 1)
           .reshape(B0, JDIM, IDIM, TG, 128)
           .transpose(0, 1, 3, 2, 4)
           .reshape(M * E))
    wi = lax.bitcast_convert_type(W, jnp.int32)
    wi = (wi + 0x7FFF + ((wi >> 16) & 1)) & jnp.int32(-65536)
    Wb = lax.bitcast_convert_type(wi, jnp.float32)
    wf = jnp.repeat(W.reshape(E), L)
    wbf = jnp.repeat(Wb.reshape(E), L)
    bf = jnp.repeat(b, L)
    mesh = plsc.VectorSubcoreMesh(core_axis_name="c", subcore_axis_name="s")
    vals, idxs = pl.kernel(
        _sc_body,
        out_type=(jax.ShapeDtypeStruct((K * M,), jnp.float32),
                  jax.ShapeDtypeStruct((K * M,), jnp.int32)),
        mesh=mesh,
        compiler_params=pltpu.CompilerParams(needs_layout_passes=False),
        scratch_types=[
            pltpu.VMEM((E * L,), jnp.float32),
            pltpu.VMEM((E * L,), jnp.float32),
            pltpu.VMEM((JDIM * L,), jnp.float32),
            pltpu.VMEM((BG * JDIM * TGB * 512,), jnp.float32),
            pltpu.VMEM((K * PLANE,), jnp.float32),
            pltpu.VMEM((K * PLANE,), jnp.int32),
            pltpu.SemaphoreType.DMA,
        ],
    )(xf, wf, wbf, bf)
    vals = (vals.reshape(K, NBG, TG, BG, 128)
                .transpose(1, 3, 2, 4, 0).reshape(B0, B1, K))
    idxs = (idxs.reshape(K, NBG, TG, BG, 128)
                .transpose(1, 3, 2, 4, 0).reshape(B0, B1, K))
    return vals, idxs


# speedup vs baseline: 1.6210x; 1.0666x over previous
import jax
import jax.numpy as jnp
from jax import lax
from jax.experimental import pallas as pl
from jax.experimental.pallas import tpu as pltpu
from jax.experimental.pallas import tpu_sc as plsc

B0, B1 = 64, 32768
M = B0 * B1
JDIM, IDIM = 5, 4
E = JDIM * IDIM
K = 3
NC, NS, L = 2, 16, 16
NW = NC * NS
TG = B1 // 128
BG = 8
NBG = B0 // BG
TGB = 4
NCH = NBG * (TG // TGB)
CPW = NCH // NW
CTOK = BG * TGB * 128
GROUPS = CTOK // L
PLANE = TGB * BG * 128


def _round_bf16(v):
    u = plsc.bitcast(v, jnp.int32)
    u = (u + 0x8000) & jnp.int32(-65536)
    return plsc.bitcast(u, jnp.float32)


def _sc_body(xf, wf, wbf, bf, vals, idxs,
             w_v, wb_v, b_v, in_v, vo_v, io_v, sem):
    cid = lax.axis_index("c")
    sid = lax.axis_index("s")
    wid = sid * NC + cid
    pltpu.sync_copy(wf, w_v)
    pltpu.sync_copy(wbf, wb_v)
    pltpu.sync_copy(bf, b_v)
    lanes = lax.iota(jnp.int32, L)
    wsf = [w_v[pl.ds(k * L, L)] for k in range(E)]
    wsb = [wb_v[pl.ds(k * L, L)] for k in range(E)]
    bs = [b_v[pl.ds(o * L, L)] for o in range(JDIM)]

    def chunk_body(c, carry):
        ci = wid * CPW + c
        bg = ci // (TG // TGB)
        tg0 = (ci % (TG // TGB)) * TGB
        copies = [pltpu.async_copy(
                      xf.at[pl.ds((((bg * BG + bl) * JDIM + j) * TG + tg0)
                                  * 512, TGB * 512)],
                      in_v.at[pl.ds((bl * JDIM + j) * (TGB * 512),
                                    TGB * 512)], sem)
                  for bl in range(BG) for j in range(JDIM)]
        for cp in copies:
            cp.wait()

        def group_body(g, carry):
            bl = g // 32
            r = g - bl * 32
            tgl = r // 8
            g16 = r - tgl * 8
            toff = g16 * L
            obase = tgl * (BG * 128) + bl * 128 + g16 * L
            s = []
            for j in range(JDIM):
                h = [_round_bf16(
                        in_v[pl.ds((bl * JDIM + j) * (TGB * 512)
                                   + tgl * 512 + i * 128 + toff, L)]
                        + wsf[j * IDIM + i])
                     for i in range(IDIM)]
                acc_sum = None
                for o in range(JDIM):
                    acc = bs[o]
                    for i in range(IDIM):
                        acc = acc + h[i] * wsb[o * IDIM + i]
                    acc = jnp.maximum(acc, 0.0)
                    acc_sum = acc if acc_sum is None else acc_sum + acc
                s.append(acc_sum)
            key = [(plsc.bitcast(s[j], jnp.int32) & jnp.int32(-8)) | (7 - j)
                   for j in range(JDIM)]

            def comp(a, b):
                return jnp.maximum(a, b), jnp.minimum(a, b)

            k0, k1, k2, k3, k4 = key
            k0, k1 = comp(k0, k1)
            k2, k3 = comp(k2, k3)
            k0, k2 = comp(k0, k2)
            k1, k3 = comp(k1, k3)
            k1, k2 = comp(k1, k2)
            k2, _ = comp(k2, k4)
            k1, k2 = comp(k1, k2)
            k0, k1 = comp(k0, k1)
            for k, kk in enumerate((k0, k1, k2)):
                vo_v[pl.ds(k * PLANE + obase, L)] = plsc.bitcast(
                    kk & jnp.int32(-8), jnp.float32)
                io_v[pl.ds(k * PLANE + obase, L)] = (kk ^ 7) & 7
            return carry

        lax.fori_loop(0, GROUPS, group_body, 0)
        dst = bg * (TG * BG * 128) + tg0 * (BG * 128)
        oc = [pltpu.async_copy(vo_v.at[pl.ds(k * PLANE, PLANE)],
                               vals.at[pl.ds(k * M + dst, PLANE)], sem)
              for k in range(K)]
        oc += [pltpu.async_copy(io_v.at[pl.ds(k * PLANE, PLANE)],
                                idxs.at[pl.ds(k * M + dst, PLANE)], sem)
               for k in range(K)]
        for cp in oc:
            cp.wait()
        return carry

    lax.fori_loop(0, CPW, chunk_body, 0)


def kernel(x, W, b):
    xf = (x.transpose(0, 2, 3, 1)
           .reshape(B0, JDIM, IDIM, TG, 128)
           .transpose(0, 1, 3, 2, 4)
           .reshape(M * E))
    wi = lax.bitcast_convert_type(W, jnp.int32)
    wi = (wi + 0x7FFF + ((wi >> 16) & 1)) & jnp.int32(-65536)
    Wb = lax.bitcast_convert_type(wi, jnp.float32)
    wf = jnp.repeat(W.reshape(E), L)
    wbf = jnp.repeat(Wb.reshape(E), L)
    bf = jnp.repeat(b, L)
    mesh = plsc.VectorSubcoreMesh(core_axis_name="c", subcore_axis_name="s")
    vals, idxs = pl.kernel(
        _sc_body,
        out_type=(jax.ShapeDtypeStruct((K * M,), jnp.float32),
                  jax.ShapeDtypeStruct((K * M,), jnp.int32)),
        mesh=mesh,
        compiler_params=pltpu.CompilerParams(needs_layout_passes=False),
        scratch_types=[
            pltpu.VMEM((E * L,), jnp.float32),
            pltpu.VMEM((E * L,), jnp.float32),
            pltpu.VMEM((JDIM * L,), jnp.float32),
            pltpu.VMEM((BG * JDIM * TGB * 512,), jnp.float32),
            pltpu.VMEM((K * PLANE,), jnp.float32),
            pltpu.VMEM((K * PLANE,), jnp.int32),
            pltpu.SemaphoreType.DMA,
        ],
    )(xf, wf, wbf, bf)
    vals = (vals.reshape(K, NBG, TG, BG, 128)
                .transpose(1, 3, 2, 4, 0).reshape(B0, B1, K))
    idxs = (idxs.reshape(K, NBG, TG, BG, 128)
                .transpose(1, 3, 2, 4, 0).reshape(B0, B1, K))
    return vals, idxs


# speedup vs baseline: 1.7385x; 1.0725x over previous
import jax
import jax.numpy as jnp
from jax import lax
from jax.experimental import pallas as pl
from jax.experimental.pallas import tpu as pltpu
from jax.experimental.pallas import tpu_sc as plsc

B0, B1 = 64, 32768
M = B0 * B1
JDIM, IDIM = 5, 4
E = JDIM * IDIM
K = 3
NC, NS, L = 2, 16, 16
NW = NC * NS
TG = B1 // 128
BG = 8
NBG = B0 // BG
TGB = 4
NCH = NBG * (TG // TGB)
CPW = NCH // NW
CTOK = BG * TGB * 128
GROUPS = CTOK // L
PLANE = TGB * BG * 128


def _round_bf16(v):
    u = plsc.bitcast(v, jnp.int32)
    u = (u + 0x8000) & jnp.int32(-65536)
    return plsc.bitcast(u, jnp.float32)


def _sc_body(xf, wf, wbf, bf, vals, idxs,
             w_v, wb_v, b_v, in_v, vo_v, io_v, sem):
    cid = lax.axis_index("c")
    sid = lax.axis_index("s")
    wid = sid * NC + cid
    pltpu.sync_copy(wf, w_v)
    pltpu.sync_copy(wbf, wb_v)
    pltpu.sync_copy(bf, b_v)
    lanes = lax.iota(jnp.int32, L)
    wsf = [w_v[pl.ds(k * L, L)] for k in range(E)]
    wsb = [wb_v[pl.ds(k * L, L)] for k in range(E)]
    bs = [b_v[pl.ds(o * L, L)] for o in range(JDIM)]

    def chunk_body(c, carry):
        ci = wid * CPW + c
        bg = ci // (TG // TGB)
        tg0 = (ci % (TG // TGB)) * TGB
        copies = [pltpu.async_copy(
                      xf.at[pl.ds((((bg * BG + bl) * JDIM + j) * TG + tg0)
                                  * 512, TGB * 512)],
                      in_v.at[pl.ds((bl * JDIM + j) * (TGB * 512),
                                    TGB * 512)], sem)
                  for bl in range(BG) for j in range(JDIM)]
        for cp in copies:
            cp.wait()

        def group_body(g, carry):
            bl = g // 32
            r = g - bl * 32
            tgl = r // 8
            g16 = r - tgl * 8
            toff = g16 * L
            obase = tgl * (BG * 128) + bl * 128 + g16 * L
            s = []
            for j in range(JDIM):
                h = [_round_bf16(
                        in_v[pl.ds((bl * JDIM + j) * (TGB * 512)
                                   + tgl * 512 + i * 128 + toff, L)]
                        + wsf[j * IDIM + i])
                     for i in range(IDIM)]
                acc_sum = None
                for o in range(JDIM):
                    acc = bs[o]
                    for i in range(IDIM):
                        acc = acc + h[i] * wsb[o * IDIM + i]
                    acc = jnp.maximum(acc, 0.0)
                    acc_sum = acc if acc_sum is None else acc_sum + acc
                s.append(acc_sum)
            key = [(plsc.bitcast(s[j], jnp.int32) & jnp.int32(-8)) | (7 - j)
                   for j in range(JDIM)]

            def comp(a, b):
                return jnp.maximum(a, b), jnp.minimum(a, b)

            k0, k1, k2, k3, k4 = key
            k0, k1 = comp(k0, k1)
            k2, k3 = comp(k2, k3)
            k0, k2 = comp(k0, k2)
            k1, k3 = comp(k1, k3)
            k1, k2 = comp(k1, k2)
            k2, _ = comp(k2, k4)
            k1, k2 = comp(k1, k2)
            k0, k1 = comp(k0, k1)
            for k, kk in enumerate((k0, k1, k2)):
                vo_v[pl.ds(k * PLANE + obase, L)] = plsc.bitcast(
                    kk & jnp.int32(-8), jnp.float32)
                io_v[pl.ds(k * PLANE + obase, L)] = (kk ^ 7) & 7
            return carry

        @plsc.parallel_loop(0, GROUPS, 1, unroll=2)
        def _(g):
            group_body(g, 0)
        dst = bg * (TG * BG * 128) + tg0 * (BG * 128)
        oc = [pltpu.async_copy(vo_v.at[pl.ds(k * PLANE, PLANE)],
                               vals.at[pl.ds(k * M + dst, PLANE)], sem)
              for k in range(K)]
        oc += [pltpu.async_copy(io_v.at[pl.ds(k * PLANE, PLANE)],
                                idxs.at[pl.ds(k * M + dst, PLANE)], sem)
               for k in range(K)]
        for cp in oc:
            cp.wait()
        return carry

    lax.fori_loop(0, CPW, chunk_body, 0)


def kernel(x, W, b):
    xf = (x.transpose(0, 2, 3, 1)
           .reshape(B0, JDIM, IDIM, TG, 128)
           .transpose(0, 1, 3, 2, 4)
           .reshape(M * E))
    wi = lax.bitcast_convert_type(W, jnp.int32)
    wi = (wi + 0x7FFF + ((wi >> 16) & 1)) & jnp.int32(-65536)
    Wb = lax.bitcast_convert_type(wi, jnp.float32)
    wf = jnp.repeat(W.reshape(E), L)
    wbf = jnp.repeat(Wb.reshape(E), L)
    bf = jnp.repeat(b, L)
    mesh = plsc.VectorSubcoreMesh(core_axis_name="c", subcore_axis_name="s")
    vals, idxs = pl.kernel(
        _sc_body,
        out_type=(jax.ShapeDtypeStruct((K * M,), jnp.float32),
                  jax.ShapeDtypeStruct((K * M,), jnp.int32)),
        mesh=mesh,
        compiler_params=pltpu.CompilerParams(needs_layout_passes=False),
        scratch_types=[
            pltpu.VMEM((E * L,), jnp.float32),
            pltpu.VMEM((E * L,), jnp.float32),
            pltpu.VMEM((JDIM * L,), jnp.float32),
            pltpu.VMEM((BG * JDIM * TGB * 512,), jnp.float32),
            pltpu.VMEM((K * PLANE,), jnp.float32),
            pltpu.VMEM((K * PLANE,), jnp.int32),
            pltpu.SemaphoreType.DMA,
        ],
    )(xf, wf, wbf, bf)
    vals = (vals.reshape(K, NBG, TG, BG, 128)
                .transpose(1, 3, 2, 4, 0).reshape(B0, B1, K))
    idxs = (idxs.reshape(K, NBG, TG, BG, 128)
                .transpose(1, 3, 2, 4, 0).reshape(B0, B1, K))
    return vals, idxs


# speedup vs baseline: 1.7437x; 1.0030x over previous
import jax
import jax.numpy as jnp
from jax import lax
from jax.experimental import pallas as pl
from jax.experimental.pallas import tpu as pltpu
from jax.experimental.pallas import tpu_sc as plsc

B0, B1 = 64, 32768
M = B0 * B1
JDIM, IDIM = 5, 4
E = JDIM * IDIM
K = 3
NC, NS, L = 2, 16, 16
NW = NC * NS
TG = B1 // 128
BG = 8
NBG = B0 // BG
TGB = 4
NCH = NBG * (TG // TGB)
CPW = NCH // NW
CTOK = BG * TGB * 128
GROUPS = CTOK // L
PLANE = TGB * BG * 128


def _round_bf16(v):
    u = plsc.bitcast(v, jnp.int32)
    u = (u + 0x8000) & jnp.int32(-65536)
    return plsc.bitcast(u, jnp.float32)


def _sc_body(xf, wf, wbf, bf, vals, idxs,
             w_v, wb_v, b_v, in_v, vo_v, io_v, sem):
    cid = lax.axis_index("c")
    sid = lax.axis_index("s")
    wid = sid * NC + cid
    pltpu.sync_copy(wf, w_v)
    pltpu.sync_copy(wbf, wb_v)
    pltpu.sync_copy(bf, b_v)
    lanes = lax.iota(jnp.int32, L)
    wsf = [w_v[pl.ds(k * L, L)] for k in range(E)]
    wsb = [wb_v[pl.ds(k * L, L)] for k in range(E)]
    bs = [b_v[pl.ds(o * L, L)] for o in range(JDIM)]

    def chunk_body(c, carry):
        ci = wid * CPW + c
        bg = ci // (TG // TGB)
        tg0 = (ci % (TG // TGB)) * TGB
        copies = [pltpu.async_copy(
                      xf.at[pl.ds((((bg * BG + bl) * JDIM + j) * TG + tg0)
                                  * 512, TGB * 512)],
                      in_v.at[pl.ds((bl * JDIM + j) * (TGB * 512),
                                    TGB * 512)], sem)
                  for bl in range(BG) for j in range(JDIM)]
        for cp in copies:
            cp.wait()

        def group_body(g, carry):
            bl = g // 32
            r = g - bl * 32
            tgl = r // 8
            g16 = r - tgl * 8
            toff = g16 * L
            obase = tgl * (BG * 128) + bl * 128 + g16 * L
            s = []
            for j in range(JDIM):
                h = [_round_bf16(
                        in_v[pl.ds((bl * JDIM + j) * (TGB * 512)
                                   + tgl * 512 + i * 128 + toff, L)]
                        + wsf[j * IDIM + i])
                     for i in range(IDIM)]
                acc_sum = None
                for o in range(JDIM):
                    acc = bs[o]
                    for i in range(IDIM):
                        acc = acc + h[i] * wsb[o * IDIM + i]
                    acc = jnp.maximum(acc, 0.0)
                    acc_sum = acc if acc_sum is None else acc_sum + acc
                s.append(acc_sum)
            key = [(plsc.bitcast(s[j], jnp.int32) & jnp.int32(-8)) | (7 - j)
                   for j in range(JDIM)]

            def comp(a, b):
                return jnp.maximum(a, b), jnp.minimum(a, b)

            k0, k1, k2, k3, k4 = key
            k0, k1 = comp(k0, k1)
            k2, k3 = comp(k2, k3)
            k0, k2 = comp(k0, k2)
            k1, k3 = comp(k1, k3)
            k1, k2 = comp(k1, k2)
            k2, _ = comp(k2, k4)
            k1, k2 = comp(k1, k2)
            k0, k1 = comp(k0, k1)
            for k, kk in enumerate((k0, k1, k2)):
                vo_v[pl.ds(k * PLANE + obase, L)] = plsc.bitcast(
                    kk & jnp.int32(-8), jnp.float32)
                io_v[pl.ds(k * PLANE + obase, L)] = (kk ^ 7) & 7
            return carry

        @plsc.parallel_loop(0, GROUPS, 1, unroll=4)
        def _(g):
            group_body(g, 0)
        dst = bg * (TG * BG * 128) + tg0 * (BG * 128)
        oc = [pltpu.async_copy(vo_v.at[pl.ds(k * PLANE, PLANE)],
                               vals.at[pl.ds(k * M + dst, PLANE)], sem)
              for k in range(K)]
        oc += [pltpu.async_copy(io_v.at[pl.ds(k * PLANE, PLANE)],
                                idxs.at[pl.ds(k * M + dst, PLANE)], sem)
               for k in range(K)]
        for cp in oc:
            cp.wait()
        return carry

    lax.fori_loop(0, CPW, chunk_body, 0)


def kernel(x, W, b):
    xf = (x.transpose(0, 2, 3, 1)
           .reshape(B0, JDIM, IDIM, TG, 128)
           .transpose(0, 1, 3, 2, 4)
           .reshape(M * E))
    wi = lax.bitcast_convert_type(W, jnp.int32)
    wi = (wi + 0x7FFF + ((wi >> 16) & 1)) & jnp.int32(-65536)
    Wb = lax.bitcast_convert_type(wi, jnp.float32)
    wf = jnp.repeat(W.reshape(E), L)
    wbf = jnp.repeat(Wb.reshape(E), L)
    bf = jnp.repeat(b, L)
    mesh = plsc.VectorSubcoreMesh(core_axis_name="c", subcore_axis_name="s")
    vals, idxs = pl.kernel(
        _sc_body,
        out_type=(jax.ShapeDtypeStruct((K * M,), jnp.float32),
                  jax.ShapeDtypeStruct((K * M,), jnp.int32)),
        mesh=mesh,
        compiler_params=pltpu.CompilerParams(needs_layout_passes=False),
        scratch_types=[
            pltpu.VMEM((E * L,), jnp.float32),
            pltpu.VMEM((E * L,), jnp.float32),
            pltpu.VMEM((JDIM * L,), jnp.float32),
            pltpu.VMEM((BG * JDIM * TGB * 512,), jnp.float32),
            pltpu.VMEM((K * PLANE,), jnp.float32),
            pltpu.VMEM((K * PLANE,), jnp.int32),
            pltpu.SemaphoreType.DMA,
        ],
    )(xf, wf, wbf, bf)
    vals = (vals.reshape(K, NBG, TG, BG, 128)
                .transpose(1, 3, 2, 4, 0).reshape(B0, B1, K))
    idxs = (idxs.reshape(K, NBG, TG, BG, 128)
                .transpose(1, 3, 2, 4, 0).reshape(B0, B1, K))
    return vals, idxs


# speedup vs baseline: 1.9935x; 1.1433x over previous
import jax
import jax.numpy as jnp
from jax import lax
from jax.experimental import pallas as pl
from jax.experimental.pallas import tpu as pltpu
from jax.experimental.pallas import tpu_sc as plsc

B0, B1 = 64, 32768
M = B0 * B1
JDIM, IDIM = 5, 4
E = JDIM * IDIM
K = 3
NC, NS, L = 2, 16, 16
NW = NC * NS
TG = B1 // 128
BG = 8
NBG = B0 // BG
TGB = 2
NCH = NBG * (TG // TGB)
CPW = NCH // NW
CTOK = BG * TGB * 128
GROUPS = CTOK // L
PLANE = TGB * BG * 128


def _round_bf16(v):
    u = plsc.bitcast(v, jnp.int32)
    u = (u + 0x8000) & jnp.int32(-65536)
    return plsc.bitcast(u, jnp.float32)


def _sc_body(xf, wf, wbf, bf, vals, idxs,
             w_v, wb_v, b_v, in_v, in_v2, vo_v, io_v, sem, sem2, sem_o):
    cid = lax.axis_index("c")
    sid = lax.axis_index("s")
    wid = sid * NC + cid
    pltpu.sync_copy(wf, w_v)
    pltpu.sync_copy(wbf, wb_v)
    pltpu.sync_copy(bf, b_v)
    lanes = lax.iota(jnp.int32, L)
    wsf = [w_v[pl.ds(k * L, L)] for k in range(E)]
    wsb = [wb_v[pl.ds(k * L, L)] for k in range(E)]
    bs = [b_v[pl.ds(o * L, L)] for o in range(JDIM)]

    def in_copies(c, buf, bsem, issue):
        ci = wid * CPW + c
        bg = ci // (TG // TGB)
        tg0 = (ci % (TG // TGB)) * TGB
        mk = pltpu.async_copy if issue else pltpu.make_async_copy
        return [mk(xf.at[pl.ds((((bg * BG + bl) * JDIM + j) * TG + tg0)
                               * 512, TGB * 512)],
                   buf.at[pl.ds((bl * JDIM + j) * (TGB * 512),
                                TGB * 512)], bsem)
                for bl in range(BG) for j in range(JDIM)]

    def chunk_compute(c, buf):
        ci = wid * CPW + c
        bg = ci // (TG // TGB)
        tg0 = (ci % (TG // TGB)) * TGB
        in_v = buf

        def group_body(g, carry):
            bl = g // 32
            r = g - bl * 32
            tgl = r // 8
            g16 = r - tgl * 8
            toff = g16 * L
            obase = tgl * (BG * 128) + bl * 128 + g16 * L
            s = []
            for j in range(JDIM):
                h = [_round_bf16(
                        in_v[pl.ds((bl * JDIM + j) * (TGB * 512)
                                   + tgl * 512 + i * 128 + toff, L)]
                        + wsf[j * IDIM + i])
                     for i in range(IDIM)]
                acc_sum = None
                for o in range(JDIM):
                    acc = bs[o]
                    for i in range(IDIM):
                        acc = acc + h[i] * wsb[o * IDIM + i]
                    acc = jnp.maximum(acc, 0.0)
                    acc_sum = acc if acc_sum is None else acc_sum + acc
                s.append(acc_sum)
            key = [(plsc.bitcast(s[j], jnp.int32) & jnp.int32(-8)) | (7 - j)
                   for j in range(JDIM)]

            def comp(a, b):
                return jnp.maximum(a, b), jnp.minimum(a, b)

            k0, k1, k2, k3, k4 = key
            k0, k1 = comp(k0, k1)
            k2, k3 = comp(k2, k3)
            k0, k2 = comp(k0, k2)
            k1, k3 = comp(k1, k3)
            k1, k2 = comp(k1, k2)
            k2, _ = comp(k2, k4)
            k1, k2 = comp(k1, k2)
            k0, k1 = comp(k0, k1)
            for k, kk in enumerate((k0, k1, k2)):
                vo_v[pl.ds(k * PLANE + obase, L)] = plsc.bitcast(
                    kk & jnp.int32(-8), jnp.float32)
                io_v[pl.ds(k * PLANE + obase, L)] = (kk ^ 7) & 7
            return carry

        @plsc.parallel_loop(0, GROUPS, 1, unroll=4)
        def _(g):
            group_body(g, 0)
        dst = bg * (TG * BG * 128) + tg0 * (BG * 128)
        oc = [pltpu.async_copy(vo_v.at[pl.ds(k * PLANE, PLANE)],
                               vals.at[pl.ds(k * M + dst, PLANE)], sem_o)
              for k in range(K)]
        oc += [pltpu.async_copy(io_v.at[pl.ds(k * PLANE, PLANE)],
                                idxs.at[pl.ds(k * M + dst, PLANE)], sem_o)
               for k in range(K)]
        for cp in oc:
            cp.wait()

    in_copies(0, in_v, sem, True)

    def pair_body(p, carry):
        c0 = p * 2
        in_copies(c0 + 1, in_v2, sem2, True)
        for cp in in_copies(c0, in_v, sem, False):
            cp.wait()
        chunk_compute(c0, in_v)

        @pl.when(p < CPW // 2 - 1)
        def _():
            in_copies(c0 + 2, in_v, sem, True)
        for cp in in_copies(c0 + 1, in_v2, sem2, False):
            cp.wait()
        chunk_compute(c0 + 1, in_v2)
        return carry

    lax.fori_loop(0, CPW // 2, pair_body, 0)


def kernel(x, W, b):
    xf = (x.transpose(0, 2, 3, 1)
           .reshape(B0, JDIM, IDIM, TG, 128)
           .transpose(0, 1, 3, 2, 4)
           .reshape(M * E))
    wi = lax.bitcast_convert_type(W, jnp.int32)
    wi = (wi + 0x7FFF + ((wi >> 16) & 1)) & jnp.int32(-65536)
    Wb = lax.bitcast_convert_type(wi, jnp.float32)
    wf = jnp.repeat(W.reshape(E), L)
    wbf = jnp.repeat(Wb.reshape(E), L)
    bf = jnp.repeat(b, L)
    mesh = plsc.VectorSubcoreMesh(core_axis_name="c", subcore_axis_name="s")
    vals, idxs = pl.kernel(
        _sc_body,
        out_type=(jax.ShapeDtypeStruct((K * M,), jnp.float32),
                  jax.ShapeDtypeStruct((K * M,), jnp.int32)),
        mesh=mesh,
        compiler_params=pltpu.CompilerParams(needs_layout_passes=False),
        scratch_types=[
            pltpu.VMEM((E * L,), jnp.float32),
            pltpu.VMEM((E * L,), jnp.float32),
            pltpu.VMEM((JDIM * L,), jnp.float32),
            pltpu.VMEM((BG * JDIM * TGB * 512,), jnp.float32),
            pltpu.VMEM((BG * JDIM * TGB * 512,), jnp.float32),
            pltpu.VMEM((K * PLANE,), jnp.float32),
            pltpu.VMEM((K * PLANE,), jnp.int32),
            pltpu.SemaphoreType.DMA,
            pltpu.SemaphoreType.DMA,
            pltpu.SemaphoreType.DMA,
        ],
    )(xf, wf, wbf, bf)
    vals = (vals.reshape(K, NBG, TG, BG, 128)
                .transpose(1, 3, 2, 4, 0).reshape(B0, B1, K))
    idxs = (idxs.reshape(K, NBG, TG, BG, 128)
                .transpose(1, 3, 2, 4, 0).reshape(B0, B1, K))
    return vals, idxs
